# baseline (speedup 1.0000x reference)
import jax
import jax.numpy as jnp
from jax.experimental import pallas as pl
from jax.experimental.pallas import tpu as pltpu

_B, _N, _D, _De, _H = 16, 128, 128, 16, 128
_M = _B * _N
_NG = _N * _De // 128
_EPR = 128 // _De


def _body(e_ref, x_ref, wbd_ref, s_ref, be_ref,
          w1_ref, b1_ref, w2_ref, b2_ref, g_ref, bt_ref,
          w3_ref, b3_ref, w4_ref, b4_ref,
          o_ref, xu_s, mf_s, a_s):
    b = pl.program_id(0)
    f32 = jnp.float32

    @pl.when(b == 0)
    def _prep():
        x = x_ref[...]
        rs = jnp.sum(x, axis=1, keepdims=True)
        m = (rs != 0.0).astype(f32)
        val = x * m
        has = m
        s = 1
        while s < _M:
            val_sh = jnp.concatenate(
                [jnp.zeros((s, _D), f32), val[:-s]], axis=0)
            has_sh = jnp.concatenate(
                [jnp.zeros((s, 1), f32), has[:-s]], axis=0)
            val = jnp.where(has > 0, val, val_sh)
            has = jnp.maximum(has, has_sh)
            s *= 2
        xu_s[...] = val
        mf_s[...] = m

    lhs = e_ref[...]
    rs8 = jax.lax.dot_general(
        lhs, s_ref[...], (((1,), (0,)), ((), ())),
        preferred_element_type=f32)
    xu = xu_s[pl.ds(b * _N, _N), :]
    xq = (jnp.concatenate([xu, xu], axis=1)
          + jnp.concatenate([be_ref[...], be_ref[...]], axis=1))
    for p in range(4):
        emb = jax.lax.dot_general(
            lhs, wbd_ref[p], (((1,), (0,)), ((), ())),
            preferred_element_type=f32)
        emb3 = emb.reshape(_N, _NG, 256)
        msg = jnp.maximum(emb3 + xq[:, None, :], 0.0)
        m0 = (rs8[:, p:p + 1] != 0.0).astype(f32).reshape(_N, _NG, 1)
        m1 = (rs8[:, p + 4:p + 5] != 0.0).astype(f32).reshape(_N, _NG, 1)
        a_s[pl.ds(b * _NG, _NG), p * _D:(p + 1) * _D] = (
            jnp.sum(msg[:, :, :128] * m0, axis=0))
        a_s[pl.ds(b * _NG, _NG), (p + 4) * _D:(p + 5) * _D] = (
            jnp.sum(msg[:, :, 128:] * m1, axis=0))

    @pl.when(b == _B - 1)
    def _post():
        af = a_s[...].reshape(_B * _NG, _EPR, _D).reshape(_M, _D)
        m = mf_s[...]
        r = jnp.concatenate([m[1:], jnp.ones((1, 1), f32)], axis=0)
        v = af
        s = 1
        while s < _M:
            v_sh = jnp.concatenate(
                [v[s:], jnp.zeros((s, _D), f32)], axis=0)
            r_sh = jnp.concatenate(
                [r[s:], jnp.ones((s, 1), f32)], axis=0)
            v = v + jnp.where(r > 0, 0.0, v_sh)
            r = jnp.maximum(r, r_sh)
            s *= 2
        h = x_ref[...] + v
        h = jnp.maximum(jnp.dot(h, w1_ref[...],
                                preferred_element_type=f32) + b1_ref[...],
                        0.0)
        h = jnp.dot(h, w2_ref[...],
                    preferred_element_type=f32) + b2_ref[...]
        hm = h * m
        sums = jnp.sum(hm.reshape(_B, _N, _H), axis=1)
        counts = jnp.sum(m.reshape(_B, _N, 1), axis=1)
        pooled = sums / jnp.maximum(counts, 1.0)
        mu = jnp.mean(pooled, axis=1, keepdims=True)
        var = jnp.mean((pooled - mu) ** 2, axis=1, keepdims=True)
        normed = ((pooled - mu) / jnp.sqrt(var + 1e-5) * g_ref[...]
                  + bt_ref[...])
        z = jnp.maximum(jnp.dot(normed, w3_ref[...],
                                preferred_element_type=f32) + b3_ref[...],
                        0.0)
        z = jnp.dot(z, w4_ref[...],
                    preferred_element_type=f32) + b4_ref[...]
        o_ref[...] = jax.nn.sigmoid(z)


def kernel(masked_X, masked_E, We, be, W1, b1, W2, b2, gamma, beta,
           W3, b3, W4, b4):
    f32 = jnp.float32
    Xf = masked_X.reshape(_M, _D)
    e2 = masked_E.reshape(_B * _N * _NG, 128)
    wbd = jnp.zeros((4, 128, 256), f32)
    for p in range(4):
        wbd = wbd.at[p, _De * p:_De * (p + 1), 0:_D].set(We)
        wbd = wbd.at[p, _De * (p + 4):_De * (p + 5), _D:2 * _D].set(We)
    smat = (jnp.arange(128)[:, None] // _De ==
            jnp.arange(_EPR)[None, :]).astype(f32)
    cmap2 = lambda b: (0, 0)
    cmap3 = lambda b: (0, 0, 0)
    score = pl.pallas_call(
        _body,
        grid=(_B,),
        in_specs=[
            pl.BlockSpec((_N * _NG, 128), lambda b: (b, 0)),
            pl.BlockSpec((_M, _D), cmap2),
            pl.BlockSpec((4, 128, 256), cmap3),
            pl.BlockSpec((128, _EPR), cmap2),
            pl.BlockSpec((1, _D), cmap2),
            pl.BlockSpec((_D, _H), cmap2),
            pl.BlockSpec((1, _H), cmap2),
            pl.BlockSpec((_H, _H), cmap2),
            pl.BlockSpec((1, _H), cmap2),
            pl.BlockSpec((1, _H), cmap2),
            pl.BlockSpec((1, _H), cmap2),
            pl.BlockSpec((_H, _H), cmap2),
            pl.BlockSpec((1, _H), cmap2),
            pl.BlockSpec((_H, 1), cmap2),
            pl.BlockSpec((1, 1), cmap2),
        ],
        out_specs=pl.BlockSpec((_B, 1), cmap2),
        out_shape=jax.ShapeDtypeStruct((_B, 1), f32),
        scratch_shapes=[
            pltpu.VMEM((_M, _D), f32),
            pltpu.VMEM((_M, 1), f32),
            pltpu.VMEM((_B * _NG, _EPR * _D), f32),
        ],
    )(e2, Xf, wbd, smat, be.reshape(1, _D),
      W1, b1.reshape(1, _H), W2, b2.reshape(1, _H),
      gamma.reshape(1, _H), beta.reshape(1, _H),
      W3, b3.reshape(1, _H), W4, b4.reshape(1, 1))
    return score


# speedup vs baseline: 1.0000x; 1.0000x over previous
import jax
import jax.numpy as jnp
from jax.experimental import pallas as pl
from jax.experimental.pallas import tpu as pltpu

_B, _N, _D, _De, _H = 16, 128, 128, 16, 128
_M = _B * _N
_NG = _N * _De // 128
_EPR = 128 // _De


def _body(e_ref, x_ref, wbd_ref, s_ref, be_ref,
          w1_ref, b1_ref, w2_ref, b2_ref, g_ref, bt_ref,
          w3_ref, b3_ref, w4_ref, b4_ref,
          o_ref, xu_s, mf_s, a_s):
    b = pl.program_id(0)
    f32 = jnp.float32

    @pl.when(b == 0)
    def _prep():
        x = x_ref[...]
        rs = jnp.sum(x, axis=1, keepdims=True)
        m = (rs != 0.0).astype(f32)
        val = x * m
        has = m
        s = 1
        while s < _M:
            val_sh = jnp.concatenate(
                [jnp.zeros((s, _D), f32), val[:-s]], axis=0)
            has_sh = jnp.concatenate(
                [jnp.zeros((s, 1), f32), has[:-s]], axis=0)
            val = jnp.where(has > 0, val, val_sh)
            has = jnp.maximum(has, has_sh)
            s *= 2
        xu_s[...] = val
        mf_s[...] = m

    lhs = e_ref[...].reshape(_N * _NG, 128)
    rs8 = jax.lax.dot_general(
        lhs, s_ref[...], (((1,), (0,)), ((), ())),
        preferred_element_type=f32)
    xu = xu_s[pl.ds(b * _N, _N), :]
    xq = (jnp.concatenate([xu, xu], axis=1)
          + jnp.concatenate([be_ref[...], be_ref[...]], axis=1))
    for p in range(4):
        emb = jax.lax.dot_general(
            lhs, wbd_ref[p], (((1,), (0,)), ((), ())),
            preferred_element_type=f32)
        emb3 = emb.reshape(_N, _NG, 256)
        msg = jnp.maximum(emb3 + xq[:, None, :], 0.0)
        m0 = (rs8[:, p:p + 1] != 0.0).astype(f32).reshape(_N, _NG, 1)
        m1 = (rs8[:, p + 4:p + 5] != 0.0).astype(f32).reshape(_N, _NG, 1)
        a_s[pl.ds(b * _NG, _NG), p * _D:(p + 1) * _D] = (
            jnp.sum(msg[:, :, :128] * m0, axis=0))
        a_s[pl.ds(b * _NG, _NG), (p + 4) * _D:(p + 5) * _D] = (
            jnp.sum(msg[:, :, 128:] * m1, axis=0))

    @pl.when(b == _B - 1)
    def _post():
        af = a_s[...].reshape(_B * _NG, _EPR, _D).reshape(_M, _D)
        m = mf_s[...]
        r = jnp.concatenate([m[1:], jnp.ones((1, 1), f32)], axis=0)
        v = af
        s = 1
        while s < _M:
            v_sh = jnp.concatenate(
                [v[s:], jnp.zeros((s, _D), f32)], axis=0)
            r_sh = jnp.concatenate(
                [r[s:], jnp.ones((s, 1), f32)], axis=0)
            v = v + jnp.where(r > 0, 0.0, v_sh)
            r = jnp.maximum(r, r_sh)
            s *= 2
        h = x_ref[...] + v
        h = jnp.maximum(jnp.dot(h, w1_ref[...],
                                preferred_element_type=f32) + b1_ref[...],
                        0.0)
        h = jnp.dot(h, w2_ref[...],
                    preferred_element_type=f32) + b2_ref[...]
        hm = h * m
        sums = jnp.sum(hm.reshape(_B, _N, _H), axis=1)
        counts = jnp.sum(m.reshape(_B, _N, 1), axis=1)
        pooled = sums / jnp.maximum(counts, 1.0)
        mu = jnp.mean(pooled, axis=1, keepdims=True)
        var = jnp.mean((pooled - mu) ** 2, axis=1, keepdims=True)
        normed = ((pooled - mu) / jnp.sqrt(var + 1e-5) * g_ref[...]
                  + bt_ref[...])
        z = jnp.maximum(jnp.dot(normed, w3_ref[...],
                                preferred_element_type=f32) + b3_ref[...],
                        0.0)
        z = jnp.dot(z, w4_ref[...],
                    preferred_element_type=f32) + b4_ref[...]
        o_ref[...] = jax.nn.sigmoid(z)


def kernel(masked_X, masked_E, We, be, W1, b1, W2, b2, gamma, beta,
           W3, b3, W4, b4):
    f32 = jnp.float32
    Xf = masked_X.reshape(_M, _D)
    e2 = masked_E.reshape(_B, _N * _NG, 128)
    wbd = jnp.zeros((4, 128, 256), f32)
    for p in range(4):
        wbd = wbd.at[p, _De * p:_De * (p + 1), 0:_D].set(We)
        wbd = wbd.at[p, _De * (p + 4):_De * (p + 5), _D:2 * _D].set(We)
    smat = (jnp.arange(128)[:, None] // _De ==
            jnp.arange(_EPR)[None, :]).astype(f32)
    cmap2 = lambda b: (0, 0)
    cmap3 = lambda b: (0, 0, 0)
    score = pl.pallas_call(
        _body,
        grid=(_B,),
        in_specs=[
            pl.BlockSpec((1, _N * _NG, 128), lambda b: (b, 0, 0)),
            pl.BlockSpec((_M, _D), cmap2),
            pl.BlockSpec((4, 128, 256), cmap3),
            pl.BlockSpec((128, _EPR), cmap2),
            pl.BlockSpec((1, _D), cmap2),
            pl.BlockSpec((_D, _H), cmap2),
            pl.BlockSpec((1, _H), cmap2),
            pl.BlockSpec((_H, _H), cmap2),
            pl.BlockSpec((1, _H), cmap2),
            pl.BlockSpec((1, _H), cmap2),
            pl.BlockSpec((1, _H), cmap2),
            pl.BlockSpec((_H, _H), cmap2),
            pl.BlockSpec((1, _H), cmap2),
            pl.BlockSpec((_H, 1), cmap2),
            pl.BlockSpec((1, 1), cmap2),
        ],
        out_specs=pl.BlockSpec((_B, 1), cmap2),
        out_shape=jax.ShapeDtypeStruct((_B, 1), f32),
        scratch_shapes=[
            pltpu.VMEM((_M, _D), f32),
            pltpu.VMEM((_M, 1), f32),
            pltpu.VMEM((_B * _NG, _EPR * _D), f32),
        ],
    )(e2, Xf, wbd, smat, be.reshape(1, _D),
      W1, b1.reshape(1, _H), W2, b2.reshape(1, _H),
      gamma.reshape(1, _H), beta.reshape(1, _H),
      W3, b3.reshape(1, _H), W4, b4.reshape(1, 1))
    return score


# speedup vs baseline: 1.7144x; 1.7143x over previous
import jax
import jax.numpy as jnp
from jax.experimental import pallas as pl
from jax.experimental.pallas import tpu as pltpu

_B, _N, _D, _De, _H = 16, 128, 128, 16, 128
_M = _B * _N
_NG = _N * _De // 128
_EPR = 128 // _De


def _body(e_ref, x_ref, wbd_ref, s_ref, be_ref,
          w1_ref, b1_ref, w2_ref, b2_ref, g_ref, bt_ref,
          w3_ref, b3_ref, w4_ref, b4_ref,
          o_ref, xu_s, mf_s, a_s):
    b = pl.program_id(0)
    f32 = jnp.float32

    @pl.when(b == 0)
    def _prep():
        x = x_ref[...]
        rs = jnp.sum(x, axis=1, keepdims=True)
        m = (rs != 0.0).astype(f32)
        val = x * m
        has = m
        s = 1
        while s < _M:
            val_sh = jnp.concatenate(
                [jnp.zeros((s, _D), f32), val[:-s]], axis=0)
            has_sh = jnp.concatenate(
                [jnp.zeros((s, 1), f32), has[:-s]], axis=0)
            val = jnp.where(has > 0, val, val_sh)
            has = jnp.maximum(has, has_sh)
            s *= 2
        xu_s[...] = val
        mf_s[...] = m

    lhs = e_ref[...].reshape(_N * _NG, 128)
    rs8 = jax.lax.dot_general(
        lhs, s_ref[...], (((1,), (0,)), ((), ())),
        preferred_element_type=f32)
    xu = xu_s[pl.ds(b * _N, _N), :]
    xq = (jnp.concatenate([xu, xu], axis=1)
          + jnp.concatenate([be_ref[...], be_ref[...]], axis=1))
    for p in range(4):
        emb = jax.lax.dot_general(
            lhs, wbd_ref[p], (((1,), (0,)), ((), ())),
            preferred_element_type=f32)
        emb3 = emb.reshape(_N, _NG, 256)
        msg = jnp.maximum(emb3 + xq[:, None, :], 0.0)
        m0 = (rs8[:, p:p + 1] != 0.0).astype(f32).reshape(_N, _NG, 1)
        m1 = (rs8[:, p + 4:p + 5] != 0.0).astype(f32).reshape(_N, _NG, 1)
        a_s[pl.ds(b * _NG, _NG), p * _D:(p + 1) * _D] = (
            jnp.sum(msg[:, :, :128] * m0, axis=0))
        a_s[pl.ds(b * _NG, _NG), (p + 4) * _D:(p + 5) * _D] = (
            jnp.sum(msg[:, :, 128:] * m1, axis=0))

    @pl.when(b == _B - 1)
    def _post():
        af = a_s[...].reshape(_B * _NG, _EPR, _D).reshape(_M, _D)
        m = mf_s[...]
        r = jnp.concatenate([m[1:], jnp.ones((1, 1), f32)], axis=0)
        v = af
        s = 1
        while s < _M:
            v_sh = jnp.concatenate(
                [v[s:], jnp.zeros((s, _D), f32)], axis=0)
            r_sh = jnp.concatenate(
                [r[s:], jnp.ones((s, 1), f32)], axis=0)
            v = v + jnp.where(r > 0, 0.0, v_sh)
            r = jnp.maximum(r, r_sh)
            s *= 2
        h = x_ref[...] + v
        h = jnp.maximum(jnp.dot(h, w1_ref[...],
                                preferred_element_type=f32) + b1_ref[...],
                        0.0)
        h = jnp.dot(h, w2_ref[...],
                    preferred_element_type=f32) + b2_ref[...]
        hm = h * m
        sums = jnp.sum(hm.reshape(_B, _N, _H), axis=1)
        counts = jnp.sum(m.reshape(_B, _N, 1), axis=1)
        pooled = sums / jnp.maximum(counts, 1.0)
        mu = jnp.mean(pooled, axis=1, keepdims=True)
        var = jnp.mean((pooled - mu) ** 2, axis=1, keepdims=True)
        normed = ((pooled - mu) / jnp.sqrt(var + 1e-5) * g_ref[...]
                  + bt_ref[...])
        z = jnp.maximum(jnp.dot(normed, w3_ref[...],
                                preferred_element_type=f32) + b3_ref[...],
                        0.0)
        z = jnp.dot(z, w4_ref[...],
                    preferred_element_type=f32) + b4_ref[...]
        o_ref[...] = jax.nn.sigmoid(z)


def kernel(masked_X, masked_E, We, be, W1, b1, W2, b2, gamma, beta,
           W3, b3, W4, b4):
    f32 = jnp.float32
    Xf = masked_X.reshape(_M, _D)
    e4 = masked_E.reshape(_B, _N, _NG, 128)
    wbd = jnp.zeros((4, 128, 256), f32)
    for p in range(4):
        wbd = wbd.at[p, _De * p:_De * (p + 1), 0:_D].set(We)
        wbd = wbd.at[p, _De * (p + 4):_De * (p + 5), _D:2 * _D].set(We)
    smat = (jnp.arange(128)[:, None] // _De ==
            jnp.arange(_EPR)[None, :]).astype(f32)
    cmap2 = lambda b: (0, 0)
    cmap3 = lambda b: (0, 0, 0)
    score = pl.pallas_call(
        _body,
        grid=(_B,),
        in_specs=[
            pl.BlockSpec((1, _N, _NG, 128), lambda b: (b, 0, 0, 0)),
            pl.BlockSpec((_M, _D), cmap2),
            pl.BlockSpec((4, 128, 256), cmap3),
            pl.BlockSpec((128, _EPR), cmap2),
            pl.BlockSpec((1, _D), cmap2),
            pl.BlockSpec((_D, _H), cmap2),
            pl.BlockSpec((1, _H), cmap2),
            pl.BlockSpec((_H, _H), cmap2),
            pl.BlockSpec((1, _H), cmap2),
            pl.BlockSpec((1, _H), cmap2),
            pl.BlockSpec((1, _H), cmap2),
            pl.BlockSpec((_H, _H), cmap2),
            pl.BlockSpec((1, _H), cmap2),
            pl.BlockSpec((_H, 1), cmap2),
            pl.BlockSpec((1, 1), cmap2),
        ],
        out_specs=pl.BlockSpec((_B, 1), cmap2),
        out_shape=jax.ShapeDtypeStruct((_B, 1), f32),
        scratch_shapes=[
            pltpu.VMEM((_M, _D), f32),
            pltpu.VMEM((_M, 1), f32),
            pltpu.VMEM((_B * _NG, _EPR * _D), f32),
        ],
    )(e4, Xf, wbd, smat, be.reshape(1, _D),
      W1, b1.reshape(1, _H), W2, b2.reshape(1, _H),
      gamma.reshape(1, _H), beta.reshape(1, _H),
      W3, b3.reshape(1, _H), W4, b4.reshape(1, 1))
    return score
